# baseline (speedup 1.0000x reference)
import functools

import numpy as np
import jax
import jax.numpy as jnp
from jax import lax
from jax.experimental import pallas as pl
from jax.experimental.pallas import tpu as pltpu
from jax.experimental.pallas import tpu_sc as plsc

RCR = 5.1
RMIN = 0.8
NRBF = 16
RCA = 3.5
AMIN = 0.8
NDIV = 4
NSEC = 4
NSPECIES = 7
ETA_R = 19.7
ETA_A = 12.5
ZETA = 14.1
SHFR = np.linspace(RMIN, RCR, NRBF + 1)[:-1].astype(np.float32)
SHFA = np.linspace(AMIN, RCA, NDIV + 1)[:-1].astype(np.float32)
SHFZ = ((np.arange(NSEC) + 0.5) * np.pi / NSEC).astype(np.float32)
CZ = np.cos(SHFZ).astype(np.float32)
SZ = np.sin(SHFZ).astype(np.float32)
LN2 = 0.6931471805599453
LOG2C = (-3.2407022, 7.1100354, -7.443873, 5.7234015,
         -2.9452062, 0.9618663, -0.18029977, 0.014778721)

KCAP = 96
NSP = NSPECIES * (NSPECIES + 1) // 2
ANG_W = NSP * NDIV * NSEC
NW = 32
APW = 320


def _triu_index(num_species):
    s1, s2 = np.triu_indices(num_species)
    pidx = np.arange(s1.shape[0])
    ret = np.zeros((num_species, num_species), dtype=np.int64)
    ret[s1, s2] = pidx
    ret[s2, s1] = pidx
    return ret


TRIU16_FLAT = (_triu_index(NSPECIES).reshape(-1) * 16).astype(np.int32)


def _cosine_cutoff(d, cutoff):
    return 0.5 * (jnp.cos(jnp.pi * d / cutoff) + 1.0) * (d <= cutoff)


def _rfv_kernel(d_ref, out_ref):
    d = d_ref[...]
    fc = 0.5 * (jnp.cos(jnp.pi / RCR * d) + 1.0) * (d <= RCR)
    for c in range(NRBF):
        x = d - float(SHFR[c])
        out_ref[c] = 0.25 * jnp.exp(-ETA_R * x * x) * fc


def _radial_fv(d):
    P = d.shape[0]
    rows = P // 128
    d2 = d.reshape(rows, 128)
    out = pl.pallas_call(
        _rfv_kernel,
        out_shape=jax.ShapeDtypeStruct((NRBF, rows, 128), jnp.float32),
    )(d2)
    return out.reshape(NRBF, P).T


def _log2_poly(m):
    p = jnp.full_like(m, LOG2C[-1])
    for c in reversed(LOG2C[:-1]):
        p = p * m + c
    return p


def _make_angular(N):
    mesh = plsc.VectorSubcoreMesh(core_axis_name="c", subcore_axis_name="s")

    @functools.partial(
        pl.kernel,
        out_type=jax.ShapeDtypeStruct((N * ANG_W,), jnp.float32),
        mesh=mesh,
        compiler_params=pltpu.CompilerParams(needs_layout_passes=False),
        scratch_types=[
            pltpu.VMEM((APW + 16,), jnp.float32),
            pltpu.VMEM((APW + 16,), jnp.float32),
            pltpu.VMEM((64,), jnp.int32),
            pltpu.VMEM((KCAP * 8,), jnp.float32),
            pltpu.VMEM((ANG_W,), jnp.float32),
        ],
    )
    def ang(packed_hbm, offs_hbm, cnts_hbm, triu_hbm, out_hbm,
            offs_v, cnts_v, triu_v, rowbuf, acc):
        w = lax.axis_index("s") * 2 + lax.axis_index("c")
        base = w * APW
        pltpu.sync_copy(offs_hbm.at[pl.ds(base, APW + 16)], offs_v)
        pltpu.sync_copy(cnts_hbm.at[pl.ds(base, APW + 16)], cnts_v)
        pltpu.sync_copy(triu_hbm, triu_v)
        iota = lax.iota(jnp.int32, 16)
        lane0 = iota == 0
        zero16 = jnp.zeros((16,), jnp.float32)
        f0 = jnp.zeros((16,), jnp.int32)
        f1c = f0 + 1
        f2c = f0 + 2
        f3c = f0 + 3
        f4c = f0 + 4
        f5c = f0 + 5
        f6c = f0 + 6

        def per_atom(i, carry):
            a = base + i

            @pl.when(a < N)
            def _():
                off_s = offs_v[pl.ds(i, 16)][0].astype(jnp.int32)
                c_s = cnts_v[pl.ds(i, 16)][0].astype(jnp.int32)
                for k in range(NSP):
                    acc[pl.ds(k * 16, 16)] = zero16

                @pl.when(c_s >= 2)
                def _():
                    pltpu.sync_copy(packed_hbm.at[pl.ds(off_s * 8, KCAP * 8)],
                                    rowbuf)

                    def jj_body(jj, c1):
                        jv = jnp.full((16,), jj * 8, jnp.int32)
                        xj = plsc.load_gather(rowbuf, [jv])
                        yj = plsc.load_gather(rowbuf, [jv + f1c])
                        zj = plsc.load_gather(rowbuf, [jv + f2c])
                        dj = plsc.load_gather(rowbuf, [jv + f3c])
                        fcj = plsc.load_gather(rowbuf, [jv + f4c])
                        oj7 = plsc.load_gather(rowbuf, [jv + f6c])

                        def v_body(v, c2):
                            ii = v * 16 + iota
                            msk = ii < jj
                            i8 = ii * 8
                            xi = plsc.load_gather(rowbuf, [i8], mask=msk)
                            yi = plsc.load_gather(rowbuf, [i8 + f1c], mask=msk)
                            zi = plsc.load_gather(rowbuf, [i8 + f2c], mask=msk)
                            di = plsc.load_gather(rowbuf, [i8 + f3c], mask=msk)
                            fci = plsc.load_gather(rowbuf, [i8 + f4c], mask=msk)
                            oi = plsc.load_gather(rowbuf, [i8 + f5c], mask=msk)
                            dot = xi * xj + yi * yj + zi * zj
                            cc = jnp.clip(0.95 * dot / (di * dj), -0.98, 0.98)
                            uu = 1.0 - cc * cc
                            bi = plsc.bitcast(uu, jnp.int32)
                            r0 = plsc.bitcast(0x5f3759df - (bi >> 1), jnp.float32)
                            r1 = r0 * (1.5 - 0.5 * uu * r0 * r0)
                            r2 = r1 * (1.5 - 0.5 * uu * r1 * r1)
                            ss = uu * r2
                            dm = 0.5 * (di + dj)
                            pref = 2.0 * fci * fcj
                            chan = (oi + oj7).astype(jnp.int32)
                            t16 = plsc.load_gather(triu_v, [chan], mask=msk)
                            f2s = []
                            for a4 in range(NDIV):
                                e2 = dm - float(SHFA[a4])
                                f2s.append(jnp.exp(-ETA_A * e2 * e2))
                            for z in range(NSEC):
                                tz = 0.5 + 0.5 * (cc * float(CZ[z]) + ss * float(SZ[z]))
                                b = plsc.bitcast(tz, jnp.int32)
                                e = ((b >> 23) - 127).astype(jnp.float32)
                                m = plsc.bitcast((b & 0x7fffff) | 0x3f800000,
                                                 jnp.float32)
                                l2 = e + _log2_poly(m)
                                f1z = jnp.exp((ZETA * LN2) * l2) * pref
                                for a4 in range(NDIV):
                                    val = f1z * f2s[a4]
                                    addr = t16 + (a4 * NSEC + z)
                                    plsc.addupdate_scatter(acc, [addr], val,
                                                           mask=msk)
                            return c2

                        nv = (jj + 15) >> 4
                        lax.fori_loop(0, nv, v_body, 0)
                        return c1

                    lax.fori_loop(1, c_s, jj_body, 0)

                pltpu.sync_copy(acc, out_hbm.at[pl.ds(a * ANG_W, ANG_W)])

            return carry

        lax.fori_loop(0, APW, per_atom, 0)

    return ang


def kernel(d_ij, r_ij, pair_indices, atom_index, number_of_atoms):
    N = atom_index.shape[0]
    P = pair_indices.shape[1]
    d = d_ij[:, 0]
    rfv = _radial_fv(d)
    species12 = atom_index[pair_indices]
    index12 = pair_indices * NSPECIES + species12[::-1]
    radial_aev = jnp.zeros((N * NSPECIES, NRBF), dtype=jnp.float32)
    radial_aev = radial_aev.at[index12.reshape(-1)].add(
        jnp.concatenate([rfv, rfv], axis=0))
    radial_aev = radial_aev.reshape(N, NSPECIES * NRBF)

    close = d <= RCA
    slot_atom = pair_indices.reshape(-1)
    slot_valid = jnp.concatenate([close, close])
    sort_key = jnp.where(slot_valid, slot_atom, N).astype(jnp.int32)
    pos = jnp.arange(2 * P, dtype=jnp.int32)
    sorted_key, order = lax.sort((sort_key, pos), num_keys=1, is_stable=True)
    minpos = jnp.full((N + 2,), 2 * P, jnp.int32).at[sorted_key].min(pos)
    off = lax.cummin(minpos, axis=0, reverse=True)[:N + 1]
    cnt = jnp.minimum(off[1:] - off[:-1], KCAP)
    in_first = order < P
    ent_pair = jnp.where(in_first, order, order - P)
    sgn = jnp.where(in_first, 1.0, -1.0)
    r4 = jnp.concatenate([r_ij, d_ij], axis=1)
    g = r4[ent_pair]
    ev = g[:, :3] * sgn[:, None]
    ed = g[:, 3]
    efc = _cosine_cutoff(ed, RCA)
    osp_all = jnp.concatenate([species12[1], species12[0]])
    eosp = osp_all[order].astype(jnp.float32)
    packed = jnp.concatenate(
        [ev, ed[:, None], efc[:, None], eosp[:, None], eosp[:, None] * 7.0,
         jnp.zeros((2 * P, 1), jnp.float32)], axis=1)
    packed = jnp.pad(packed, ((0, KCAP), (0, 0))).reshape(-1)
    pad_to = NW * APW + 16
    offs_f = jnp.pad(off[:N].astype(jnp.float32), (0, pad_to - N))
    cnts_f = jnp.pad(cnt.astype(jnp.float32), (0, pad_to - N))
    triu16 = jnp.asarray(np.pad(TRIU16_FLAT, (0, 64 - TRIU16_FLAT.shape[0])))

    angular_aev = _make_angular(N)(packed, offs_f, cnts_f,
                                   triu16).reshape(N, ANG_W)
    aevs = jnp.concatenate([radial_aev, angular_aev], axis=-1)
    return (atom_index, aevs)


# speedup vs baseline: 1.0684x; 1.0684x over previous
import functools

import numpy as np
import jax
import jax.numpy as jnp
from jax import lax
from jax.experimental import pallas as pl
from jax.experimental.pallas import tpu as pltpu
from jax.experimental.pallas import tpu_sc as plsc

RCR = 5.1
RMIN = 0.8
NRBF = 16
RCA = 3.5
AMIN = 0.8
NDIV = 4
NSEC = 4
NSPECIES = 7
ETA_R = 19.7
ETA_A = 12.5
ZETA = 14.1
SHFR = np.linspace(RMIN, RCR, NRBF + 1)[:-1].astype(np.float32)
SHFA = np.linspace(AMIN, RCA, NDIV + 1)[:-1].astype(np.float32)
SHFZ = ((np.arange(NSEC) + 0.5) * np.pi / NSEC).astype(np.float32)
CZ = np.cos(SHFZ).astype(np.float32)
SZ = np.sin(SHFZ).astype(np.float32)
LN2 = 0.6931471805599453
LOG2C = (-3.2407022, 7.1100354, -7.443873, 5.7234015,
         -2.9452062, 0.9618663, -0.18029977, 0.014778721)

KCAP = 96
NSP = NSPECIES * (NSPECIES + 1) // 2
ANG_W = NSP * NDIV * NSEC
NW = 32
APW = 320


def _triu_index(num_species):
    s1, s2 = np.triu_indices(num_species)
    pidx = np.arange(s1.shape[0])
    ret = np.zeros((num_species, num_species), dtype=np.int64)
    ret[s1, s2] = pidx
    ret[s2, s1] = pidx
    return ret


TRIU16_FLAT = (_triu_index(NSPECIES).reshape(-1) * 16).astype(np.int32)


def _cosine_cutoff(d, cutoff):
    return 0.5 * (jnp.cos(jnp.pi * d / cutoff) + 1.0) * (d <= cutoff)


def _rfv_kernel(d_ref, out_ref):
    d = d_ref[...]
    fc = 0.5 * (jnp.cos(jnp.pi / RCR * d) + 1.0) * (d <= RCR)
    for c in range(NRBF):
        x = d - float(SHFR[c])
        out_ref[c] = 0.25 * jnp.exp(-ETA_R * x * x) * fc


def _radial_fv(d):
    P = d.shape[0]
    rows = P // 128
    d2 = d.reshape(rows, 128)
    out = pl.pallas_call(
        _rfv_kernel,
        out_shape=jax.ShapeDtypeStruct((NRBF, rows, 128), jnp.float32),
    )(d2)
    return out.reshape(NRBF, P).T


def _log2_poly(m):
    p = jnp.full_like(m, LOG2C[-1])
    for c in reversed(LOG2C[:-1]):
        p = p * m + c
    return p


def _make_angular(N):
    mesh = plsc.VectorSubcoreMesh(core_axis_name="c", subcore_axis_name="s")

    @functools.partial(
        pl.kernel,
        out_type=jax.ShapeDtypeStruct((N * ANG_W,), jnp.float32),
        mesh=mesh,
        compiler_params=pltpu.CompilerParams(needs_layout_passes=False),
        scratch_types=[
            pltpu.VMEM((APW + 16,), jnp.float32),
            pltpu.VMEM((APW + 16,), jnp.float32),
            pltpu.VMEM((64,), jnp.int32),
            pltpu.VMEM((KCAP * 8,), jnp.float32),
            pltpu.VMEM((ANG_W,), jnp.float32),
        ],
    )
    def ang(packed_hbm, offs_hbm, cnts_hbm, triu_hbm, out_hbm,
            offs_v, cnts_v, triu_v, rowbuf, acc):
        w = lax.axis_index("s") * 2 + lax.axis_index("c")
        base = w * APW
        pltpu.sync_copy(offs_hbm.at[pl.ds(base, APW + 16)], offs_v)
        pltpu.sync_copy(cnts_hbm.at[pl.ds(base, APW + 16)], cnts_v)
        pltpu.sync_copy(triu_hbm, triu_v)
        iota = lax.iota(jnp.int32, 16)
        lane0 = iota == 0
        zero16 = jnp.zeros((16,), jnp.float32)
        f0 = jnp.zeros((16,), jnp.int32)
        f1c = f0 + 1
        f2c = f0 + 2
        f3c = f0 + 3
        f4c = f0 + 4
        f5c = f0 + 5
        f6c = f0 + 6

        def per_atom(i, carry):
            a = base + i

            @pl.when(a < N)
            def _():
                off_s = offs_v[pl.ds(i, 16)][0].astype(jnp.int32)
                c_s = cnts_v[pl.ds(i, 16)][0].astype(jnp.int32)
                for k in range(NSP):
                    acc[pl.ds(k * 16, 16)] = zero16

                @pl.when(c_s >= 2)
                def _():
                    pltpu.sync_copy(packed_hbm.at[pl.ds(off_s * 8, KCAP * 8)],
                                    rowbuf)

                    def jj_body(jj, c1):
                        jv = jnp.full((16,), jj * 8, jnp.int32)
                        xj = plsc.load_gather(rowbuf, [jv])
                        yj = plsc.load_gather(rowbuf, [jv + f1c])
                        zj = plsc.load_gather(rowbuf, [jv + f2c])
                        dj = plsc.load_gather(rowbuf, [jv + f3c])
                        fcj = plsc.load_gather(rowbuf, [jv + f4c])
                        oj7 = plsc.load_gather(rowbuf, [jv + f6c])

                        def v_body(v, c2):
                            ii = v * 16 + iota
                            msk = ii < jj
                            i8 = ii * 8
                            xi = plsc.load_gather(rowbuf, [i8], mask=msk)
                            yi = plsc.load_gather(rowbuf, [i8 + f1c], mask=msk)
                            zi = plsc.load_gather(rowbuf, [i8 + f2c], mask=msk)
                            di = plsc.load_gather(rowbuf, [i8 + f3c], mask=msk)
                            fci = plsc.load_gather(rowbuf, [i8 + f4c], mask=msk)
                            oi = plsc.load_gather(rowbuf, [i8 + f5c], mask=msk)
                            dot = xi * xj + yi * yj + zi * zj
                            cc = jnp.clip(0.95 * dot / (di * dj), -0.98, 0.98)
                            uu = 1.0 - cc * cc
                            bi = plsc.bitcast(uu, jnp.int32)
                            r0 = plsc.bitcast(0x5f3759df - (bi >> 1), jnp.float32)
                            r1 = r0 * (1.5 - 0.5 * uu * r0 * r0)
                            r2 = r1 * (1.5 - 0.5 * uu * r1 * r1)
                            ss = uu * r2
                            dm = 0.5 * (di + dj)
                            pref = 2.0 * fci * fcj
                            chan = (oi + oj7).astype(jnp.int32)
                            t16 = plsc.load_gather(triu_v, [chan], mask=msk)
                            f2s = []
                            for a4 in range(NDIV):
                                e2 = dm - float(SHFA[a4])
                                f2s.append(jnp.exp(-ETA_A * e2 * e2))
                            for z in range(NSEC):
                                tz = 0.5 + 0.5 * (cc * float(CZ[z]) + ss * float(SZ[z]))
                                b = plsc.bitcast(tz, jnp.int32)
                                e = ((b >> 23) - 127).astype(jnp.float32)
                                m = plsc.bitcast((b & 0x7fffff) | 0x3f800000,
                                                 jnp.float32)
                                l2 = e + _log2_poly(m)
                                f1z = jnp.exp((ZETA * LN2) * l2) * pref
                                for a4 in range(NDIV):
                                    val = f1z * f2s[a4]
                                    addr = t16 + (a4 * NSEC + z)
                                    plsc.addupdate_scatter(acc, [addr], val,
                                                           mask=msk)
                            return c2

                        nv = (jj + 15) >> 4
                        lax.fori_loop(0, nv, v_body, 0)
                        return c1

                    lax.fori_loop(1, c_s, jj_body, 0)

                pltpu.sync_copy(acc, out_hbm.at[pl.ds(a * ANG_W, ANG_W)])

            return carry

        lax.fori_loop(0, APW, per_atom, 0)

    return ang


def kernel(d_ij, r_ij, pair_indices, atom_index, number_of_atoms):
    N = atom_index.shape[0]
    P = pair_indices.shape[1]
    d = d_ij[:, 0]
    rfv = _radial_fv(d)
    species12 = atom_index[pair_indices]
    index12 = pair_indices * NSPECIES + species12[::-1]
    radial_aev = jnp.zeros((N * NSPECIES, NRBF), dtype=jnp.float32)
    radial_aev = radial_aev.at[index12[0]].add(rfv)
    radial_aev = radial_aev.at[index12[1]].add(rfv)
    radial_aev = radial_aev.reshape(N, NSPECIES * NRBF)

    close = d <= RCA
    slot_atom = pair_indices.reshape(-1)
    slot_valid = jnp.concatenate([close, close])
    sort_key = jnp.where(slot_valid, slot_atom, N).astype(jnp.int32)
    pos = jnp.arange(2 * P, dtype=jnp.int32)
    sorted_key, order = lax.sort((sort_key, pos), num_keys=1, is_stable=True)
    minpos = jnp.full((N + 2,), 2 * P, jnp.int32).at[sorted_key].min(pos)
    off = lax.cummin(minpos, axis=0, reverse=True)[:N + 1]
    cnt = jnp.minimum(off[1:] - off[:-1], KCAP)
    in_first = order < P
    ent_pair = jnp.where(in_first, order, order - P)
    sgn = jnp.where(in_first, 1.0, -1.0)
    r4 = jnp.concatenate([r_ij, d_ij], axis=1)
    g = r4[ent_pair]
    ev = g[:, :3] * sgn[:, None]
    ed = g[:, 3]
    efc = _cosine_cutoff(ed, RCA)
    osp_all = jnp.concatenate([species12[1], species12[0]])
    eosp = osp_all[order].astype(jnp.float32)
    packed = jnp.concatenate(
        [ev, ed[:, None], efc[:, None], eosp[:, None], eosp[:, None] * 7.0,
         jnp.zeros((2 * P, 1), jnp.float32)], axis=1)
    packed = jnp.pad(packed, ((0, KCAP), (0, 0))).reshape(-1)
    pad_to = NW * APW + 16
    offs_f = jnp.pad(off[:N].astype(jnp.float32), (0, pad_to - N))
    cnts_f = jnp.pad(cnt.astype(jnp.float32), (0, pad_to - N))
    triu16 = jnp.asarray(np.pad(TRIU16_FLAT, (0, 64 - TRIU16_FLAT.shape[0])))

    angular_aev = _make_angular(N)(packed, offs_f, cnts_f,
                                   triu16).reshape(N, ANG_W)
    aevs = jnp.concatenate([radial_aev, angular_aev], axis=-1)
    return (atom_index, aevs)
